# Initial kernel scaffold; baseline (speedup 1.0000x reference)
import functools
import math

import jax
import jax.numpy as jnp
from jax import lax
from jax.experimental import pallas as pl
from jax.experimental.pallas import tpu as pltpu
from jax.experimental.pallas import tpu_sc as plsc

_NUM_CORES = 2
_NUM_SUBCORES = 16
_NUM_WORKERS = _NUM_CORES * _NUM_SUBCORES

_CHUNK = 128
_BATCH = 1024
_K = _BATCH // _CHUNK
_NBUF = 2


def _scale_body(t_ref, o_ref, *, scale):
    o_ref[...] = t_ref[...] * scale


def _scale_table(table, scale):
    v, d = table.shape
    blk = 12500
    grid = v // blk
    return pl.pallas_call(
        functools.partial(_scale_body, scale=scale),
        out_shape=jax.ShapeDtypeStruct((v, d), table.dtype),
        grid=(grid,),
        in_specs=[pl.BlockSpec((blk, d), lambda i: (i, 0))],
        out_specs=pl.BlockSpec((blk, d), lambda i: (i, 0)),
    )(table)


def _make_gather_kernel(num_rows, d):
    assert num_rows % (_NUM_WORKERS * _BATCH) == 0
    rows_per_worker = num_rows // _NUM_WORKERS
    nbatch = rows_per_worker // _BATCH
    assert nbatch % _NBUF == 0
    chunk_rows_per_worker = rows_per_worker // _CHUNK

    mesh = plsc.VectorSubcoreMesh(
        core_axis_name="c",
        subcore_axis_name="s",
        num_cores=_NUM_CORES,
        num_subcores=_NUM_SUBCORES,
    )

    @functools.partial(
        pl.kernel,
        out_type=jax.ShapeDtypeStruct((num_rows, d), jnp.float32),
        mesh=mesh,
        scratch_types=[
            pltpu.VMEM((_NBUF, _K, _CHUNK), jnp.int32),
            pltpu.VMEM((_NBUF, _BATCH, d), jnp.float32),
            pltpu.SemaphoreType.DMA((_NBUF,)),
            pltpu.SemaphoreType.DMA((_NBUF,)),
            pltpu.SemaphoreType.DMA((_NBUF,)),
        ],
    )
    def gather_kernel(table_hbm, tok_hbm, out_hbm, idx_v, rows_v, sem_idx,
                      sem_g, sem_st):
        wid = lax.axis_index("s") * _NUM_CORES + lax.axis_index("c")
        tok_base = wid * chunk_rows_per_worker
        out_base = wid * rows_per_worker

        def idx_copy(b, s):
            return pltpu.make_async_copy(
                tok_hbm.at[pl.ds(tok_base + b * _K, _K)],
                idx_v.at[s],
                sem_idx.at[s],
            )

        def gather_copy(s, j):
            return pltpu.make_async_copy(
                table_hbm.at[idx_v.at[s].at[j]],
                rows_v.at[s].at[pl.ds(j * _CHUNK, _CHUNK)],
                sem_g.at[s],
            )

        def store_copy(b, s):
            return pltpu.make_async_copy(
                rows_v.at[s],
                out_hbm.at[pl.ds(out_base + b * _BATCH, _BATCH)],
                sem_st.at[s],
            )

        for s in range(_NBUF):
            idx_copy(s, s).start()

        @pl.loop(0, nbatch // _NBUF)
        def _(g):
            for s in range(_NBUF):
                b = g * _NBUF + s
                idx_copy(b, s).wait()

                @pl.when(b >= _NBUF)
                def _():
                    store_copy(b - _NBUF, s).wait()

                for j in range(_K):
                    gather_copy(s, j).start()
                for j in range(_K):
                    gather_copy(s, j).wait()

                @pl.when(b + _NBUF < nbatch)
                def _():
                    idx_copy(b + _NBUF, s).start()

                store_copy(b, s).start()

        for s in range(_NBUF):
            store_copy(nbatch - _NBUF + s, s).wait()

    return gather_kernel


def kernel(tokens, table):
    v, d = table.shape
    num_rows = tokens.size
    scaled = _scale_table(table, math.sqrt(d))
    tok2d = tokens.reshape(num_rows // _CHUNK, _CHUNK)
    out = _make_gather_kernel(num_rows, d)(scaled, tok2d)
    return out.reshape(*tokens.shape, d)


# speedup vs baseline: 6.3939x; 6.3939x over previous
import functools
import math

import jax
import jax.numpy as jnp
from jax import lax
from jax.experimental import pallas as pl
from jax.experimental.pallas import tpu as pltpu
from jax.experimental.pallas import tpu_sc as plsc

_NUM_CORES = 2
_NUM_SUBCORES = 16
_NUM_WORKERS = _NUM_CORES * _NUM_SUBCORES

_CHUNK = 128
_BATCH = 1024
_K = _BATCH // _CHUNK
_NBUF = 2


def _scale_body(t_ref, o_ref, *, scale):
    o_ref[...] = t_ref[...] * scale


def _scale_table(table, scale):
    v, d = table.shape
    cols = 512
    wide = table.reshape(v * d // cols, cols)
    scaled = pl.pallas_call(
        functools.partial(_scale_body, scale=scale),
        out_shape=jax.ShapeDtypeStruct(wide.shape, table.dtype),
    )(wide)
    return scaled.reshape(v, d)


def _make_gather_kernel(num_rows, d):
    assert num_rows % (_NUM_WORKERS * _BATCH) == 0
    rows_per_worker = num_rows // _NUM_WORKERS
    nbatch = rows_per_worker // _BATCH
    assert nbatch % _NBUF == 0
    chunk_rows_per_worker = rows_per_worker // _CHUNK

    mesh = plsc.VectorSubcoreMesh(
        core_axis_name="c",
        subcore_axis_name="s",
        num_cores=_NUM_CORES,
        num_subcores=_NUM_SUBCORES,
    )

    @functools.partial(
        pl.kernel,
        out_type=jax.ShapeDtypeStruct((num_rows, d), jnp.float32),
        mesh=mesh,
        scratch_types=[
            pltpu.VMEM((_NBUF, _K, _CHUNK), jnp.int32),
            pltpu.VMEM((_NBUF, _BATCH, d), jnp.float32),
            pltpu.SemaphoreType.DMA((_NBUF,)),
            pltpu.SemaphoreType.DMA((_NBUF,)),
            pltpu.SemaphoreType.DMA((_NBUF,)),
        ],
        compiler_params=pltpu.CompilerParams(use_tc_tiling_on_sc=False),
    )
    def gather_kernel(table_hbm, tok_hbm, out_hbm, idx_v, rows_v, sem_idx,
                      sem_g, sem_st):
        wid = lax.axis_index("s") * _NUM_CORES + lax.axis_index("c")
        tok_base = wid * chunk_rows_per_worker
        out_base = wid * rows_per_worker

        def idx_copy(b, s):
            return pltpu.make_async_copy(
                tok_hbm.at[pl.ds(tok_base + b * _K, _K)],
                idx_v.at[s],
                sem_idx.at[s],
            )

        def gather_copy(s, j):
            return pltpu.make_async_copy(
                table_hbm.at[idx_v.at[s].at[j]],
                rows_v.at[s].at[pl.ds(j * _CHUNK, _CHUNK)],
                sem_g.at[s],
            )

        def store_copy(b, s):
            return pltpu.make_async_copy(
                rows_v.at[s],
                out_hbm.at[pl.ds(out_base + b * _BATCH, _BATCH)],
                sem_st.at[s],
            )

        for s in range(_NBUF):
            idx_copy(s, s).start()

        @pl.loop(0, nbatch // _NBUF)
        def _(g):
            for s in range(_NBUF):
                b = g * _NBUF + s
                idx_copy(b, s).wait()

                @pl.when(b >= _NBUF)
                def _():
                    store_copy(b - _NBUF, s).wait()

                for j in range(_K):
                    gather_copy(s, j).start()
                for j in range(_K):
                    gather_copy(s, j).wait()

                @pl.when(b + _NBUF < nbatch)
                def _():
                    idx_copy(b + _NBUF, s).start()

                store_copy(b, s).start()

        for s in range(_NBUF):
            store_copy(nbatch - _NBUF + s, s).wait()

    return gather_kernel


def kernel(tokens, table):
    v, d = table.shape
    num_rows = tokens.size
    scaled = _scale_table(table, math.sqrt(d))
    tok2d = tokens.reshape(num_rows // _CHUNK, _CHUNK)
    out = _make_gather_kernel(num_rows, d)(scaled, tok2d)
    return out.reshape(*tokens.shape, d)
